# baseline (speedup 1.0000x reference)
import jax
import jax.numpy as jnp
from jax.experimental import pallas as pl
from jax.experimental.pallas import tpu as pltpu

H1 = W1 = 28
H2 = W2 = 14
H3 = W3 = 7
C1 = 5
NCLASS = 10
K1 = H1 * W1
M1 = C1 * H2 * W2
M2 = C1 * H3 * W3


def _pool_indicator(n_out, n_in, parity, dtype):
    d = jnp.arange(3)[:, None, None]
    o = jnp.arange(n_out)[None, :, None]
    i = jnp.arange(n_in)[None, None, :]
    return (i == 2 * o + parity + d - 1).astype(dtype)


def _cnn_kernel(x_ref,
                a1_00, a1_01, a1_10, a1_11,
                a2_00, a2_01, a2_10, a2_11,
                b1_ref, b2_ref, wf_ref, bf_ref,
                out_ref):
    f32 = jnp.float32
    xb = x_ref[...]

    def mm(a_ref, b):
        return jnp.dot(a_ref[...], b, preferred_element_type=f32)

    u = jnp.maximum(jnp.maximum(mm(a1_00, xb), mm(a1_01, xb)),
                    jnp.maximum(mm(a1_10, xb), mm(a1_11, xb)))
    p1 = jnp.maximum(u + b1_ref[...], 0.0)

    v = jnp.maximum(jnp.maximum(mm(a2_00, p1), mm(a2_01, p1)),
                    jnp.maximum(mm(a2_10, p1), mm(a2_11, p1)))
    p2 = jnp.maximum(v + b2_ref[...], 0.0)

    logits = mm(wf_ref, p2) + bf_ref[...]
    m = jnp.max(logits, axis=0, keepdims=True)
    shifted = logits - m
    lse = jnp.log(jnp.sum(jnp.exp(shifted), axis=0, keepdims=True))
    out_ref[...] = shifted - lse


def kernel(x, w1, b1, w2, b2, wf, bf):
    f32 = jnp.float32
    N = x.shape[0]
    BN = 256
    n_blocks = pl.cdiv(N, BN)
    n_pad = n_blocks * BN

    w1r = w1.reshape(C1, 3, 3).astype(f32)
    w2r = w2.astype(f32)
    parities = [(0, 0), (0, 1), (1, 0), (1, 1)]
    a1 = [jnp.einsum('cij,iyh,jxw->cyxhw', w1r,
                     _pool_indicator(H2, H1, py, f32),
                     _pool_indicator(W2, W1, px, f32)).reshape(M1, K1)
          for (py, px) in parities]
    a2 = [jnp.einsum('abij,iyh,jxw->ayxbhw', w2r,
                     _pool_indicator(H3, H2, py, f32),
                     _pool_indicator(W3, W2, px, f32)).reshape(M2, M1)
          for (py, px) in parities]
    b1c = jnp.repeat(b1.astype(f32), H2 * W2).reshape(M1, 1)
    b2c = jnp.repeat(b2.astype(f32), H3 * W3).reshape(M2, 1)
    wff = wf.astype(f32)
    bfc = bf.reshape(NCLASS, 1).astype(f32)

    xr = x.reshape(N, K1).astype(f32)
    if n_pad != N:
        xr = jnp.pad(xr, ((0, n_pad - N), (0, 0)))
    xt = xr.T

    out = pl.pallas_call(
        _cnn_kernel,
        out_shape=jax.ShapeDtypeStruct((NCLASS, n_pad), f32),
        grid=(n_blocks,),
        in_specs=[
            pl.BlockSpec((K1, BN), lambda n: (0, n)),
            pl.BlockSpec((M1, K1), lambda n: (0, 0)),
            pl.BlockSpec((M1, K1), lambda n: (0, 0)),
            pl.BlockSpec((M1, K1), lambda n: (0, 0)),
            pl.BlockSpec((M1, K1), lambda n: (0, 0)),
            pl.BlockSpec((M2, M1), lambda n: (0, 0)),
            pl.BlockSpec((M2, M1), lambda n: (0, 0)),
            pl.BlockSpec((M2, M1), lambda n: (0, 0)),
            pl.BlockSpec((M2, M1), lambda n: (0, 0)),
            pl.BlockSpec((M1, 1), lambda n: (0, 0)),
            pl.BlockSpec((M2, 1), lambda n: (0, 0)),
            pl.BlockSpec((NCLASS, M2), lambda n: (0, 0)),
            pl.BlockSpec((NCLASS, 1), lambda n: (0, 0)),
        ],
        out_specs=pl.BlockSpec((NCLASS, BN), lambda n: (0, n)),
        compiler_params=pltpu.CompilerParams(
            dimension_semantics=("parallel",)),
    )(xt, *a1, *a2, b1c, b2c, wff, bfc)

    return jnp.transpose(out[:, :N])


# speedup vs baseline: 1.0131x; 1.0131x over previous
import jax
import jax.numpy as jnp
from jax.experimental import pallas as pl
from jax.experimental.pallas import tpu as pltpu

H1 = W1 = 28
H2 = W2 = 14
H3 = W3 = 7
C1 = 5
NCLASS = 10
K1 = H1 * W1
M1 = C1 * H2 * W2
M2 = C1 * H3 * W3


def _pool_indicator(n_out, n_in, parity, dtype):
    d = jnp.arange(3)[:, None, None]
    o = jnp.arange(n_out)[None, :, None]
    i = jnp.arange(n_in)[None, None, :]
    return (i == 2 * o + parity + d - 1).astype(dtype)


def _cnn_kernel(x_ref,
                a1_00, a1_01, a1_10, a1_11,
                a2_00, a2_01, a2_10, a2_11,
                b1_ref, b2_ref, wf_ref, bf_ref,
                out_ref):
    f32 = jnp.float32
    xb = x_ref[...]

    def mm(a_ref, b):
        return jnp.dot(a_ref[...], b, preferred_element_type=f32)

    u = jnp.maximum(jnp.maximum(mm(a1_00, xb), mm(a1_01, xb)),
                    jnp.maximum(mm(a1_10, xb), mm(a1_11, xb)))
    p1 = jnp.maximum(u + b1_ref[...], 0.0)

    v = jnp.maximum(jnp.maximum(mm(a2_00, p1), mm(a2_01, p1)),
                    jnp.maximum(mm(a2_10, p1), mm(a2_11, p1)))
    p2 = jnp.maximum(v + b2_ref[...], 0.0)

    logits = mm(wf_ref, p2) + bf_ref[...]
    m = jnp.max(logits, axis=0, keepdims=True)
    shifted = logits - m
    lse = jnp.log(jnp.sum(jnp.exp(shifted), axis=0, keepdims=True))
    out_ref[...] = shifted - lse


def kernel(x, w1, b1, w2, b2, wf, bf):
    f32 = jnp.float32
    N = x.shape[0]
    BN = 1024
    n_blocks = pl.cdiv(N, BN)
    n_pad = n_blocks * BN

    w1r = w1.reshape(C1, 3, 3).astype(f32)
    w2r = w2.astype(f32)
    parities = [(0, 0), (0, 1), (1, 0), (1, 1)]
    a1 = [jnp.einsum('cij,iyh,jxw->cyxhw', w1r,
                     _pool_indicator(H2, H1, py, f32),
                     _pool_indicator(W2, W1, px, f32)).reshape(M1, K1)
          for (py, px) in parities]
    a2 = [jnp.einsum('abij,iyh,jxw->ayxbhw', w2r,
                     _pool_indicator(H3, H2, py, f32),
                     _pool_indicator(W3, W2, px, f32)).reshape(M2, M1)
          for (py, px) in parities]
    b1c = jnp.repeat(b1.astype(f32), H2 * W2).reshape(M1, 1)
    b2c = jnp.repeat(b2.astype(f32), H3 * W3).reshape(M2, 1)
    wff = wf.astype(f32)
    bfc = bf.reshape(NCLASS, 1).astype(f32)

    xr = x.reshape(N, K1).astype(f32)
    if n_pad != N:
        xr = jnp.pad(xr, ((0, n_pad - N), (0, 0)))
    xt = xr.T

    out = pl.pallas_call(
        _cnn_kernel,
        out_shape=jax.ShapeDtypeStruct((NCLASS, n_pad), f32),
        grid=(n_blocks,),
        in_specs=[
            pl.BlockSpec((K1, BN), lambda n: (0, n)),
            pl.BlockSpec((M1, K1), lambda n: (0, 0)),
            pl.BlockSpec((M1, K1), lambda n: (0, 0)),
            pl.BlockSpec((M1, K1), lambda n: (0, 0)),
            pl.BlockSpec((M1, K1), lambda n: (0, 0)),
            pl.BlockSpec((M2, M1), lambda n: (0, 0)),
            pl.BlockSpec((M2, M1), lambda n: (0, 0)),
            pl.BlockSpec((M2, M1), lambda n: (0, 0)),
            pl.BlockSpec((M2, M1), lambda n: (0, 0)),
            pl.BlockSpec((M1, 1), lambda n: (0, 0)),
            pl.BlockSpec((M2, 1), lambda n: (0, 0)),
            pl.BlockSpec((NCLASS, M2), lambda n: (0, 0)),
            pl.BlockSpec((NCLASS, 1), lambda n: (0, 0)),
        ],
        out_specs=pl.BlockSpec((NCLASS, BN), lambda n: (0, n)),
        compiler_params=pltpu.CompilerParams(
            dimension_semantics=("parallel",)),
    )(xt, *a1, *a2, b1c, b2c, wff, bfc)

    return jnp.transpose(out[:, :N])
